# baseline (speedup 1.0000x reference)
import jax
import jax.numpy as jnp
from jax import lax
from jax.experimental import pallas as pl
from jax.experimental.pallas import tpu as pltpu

_VMEM_LIMIT = 96 * 1024 * 1024


def _pick_tm(m, cap=6400):
    start = min(cap, m)
    start -= start % 8
    for tm in range(start, 7, -8):
        if m % tm == 0:
            return tm
    return m


def _fold(v, pk, c):
    out = v[:, 0:c]
    for j in range(1, pk):
        out = out + v[:, j * c:(j + 1) * c]
    return out


def _dup(v, pk):
    return jnp.concatenate([v] * pk, axis=1) if pk > 1 else v


def _stat_rows(y, ch):
    s = jnp.sum(y, axis=0, keepdims=True)
    q = jnp.sum(y * y, axis=0, keepdims=True)
    row = lax.broadcasted_iota(jnp.int32, (8, ch), 0)
    return jnp.where(row == 0, s, jnp.where(row == 1, q, 0.0))


def _affine_from_stats(st, gamma, beta, count, eps):
    s = jnp.sum(st[:, 0, :], axis=0, keepdims=True)
    q = jnp.sum(st[:, 1, :], axis=0, keepdims=True)
    mean = s * (1.0 / count)
    var = jnp.maximum(q * (1.0 / count) - mean * mean, 0.0)
    scale = gamma * lax.rsqrt(var + eps)
    shift = beta - mean * scale
    return scale, shift


def _conv1_kernel(x_ref, w_ref, y_ref, stat_ref):
    xb = x_ref[...].astype(jnp.bfloat16)
    wb = w_ref[...].astype(jnp.bfloat16)
    y = jnp.dot(xb, wb, preferred_element_type=jnp.float32)
    y_ref[...] = y.astype(jnp.bfloat16)
    stat_ref[0] = _stat_rows(y, y.shape[1])


def _make_conv3_kernel(H, W, M, eps, pk):
    def body(y1_ref, st1_ref, g1_ref, b1_ref, w2_ref, y2_ref, stat_ref,
             pad_ref, buf_ref):
        c = y1_ref.shape[-1]
        stp = st1_ref[...]
        st = jnp.stack([_fold(stp[:, 0, :], pk, c),
                        _fold(stp[:, 1, :], pk, c)], axis=1)
        sc, sh = _affine_from_stats(st, g1_ref[...], b1_ref[...], M, eps)
        a = jnp.maximum(y1_ref[0].astype(jnp.float32) * sc + sh, 0.0)
        zc = jnp.zeros((W + 2, c), jnp.bfloat16)
        pad_ref[0] = zc
        pad_ref[H + 1] = zc
        zr = jnp.zeros((H + 2, c), jnp.bfloat16)
        pad_ref[:, 0, :] = zr
        pad_ref[:, W + 1, :] = zr
        pad_ref[pl.ds(1, H), pl.ds(1, W), :] = (
            a.reshape(H, W, c).astype(jnp.bfloat16))

        for kw in range(3):
            buf_ref[:, :, pl.ds(kw * c, c)] = pad_ref[:, pl.ds(kw, W), :]

        acc = jnp.zeros((H * W, c), jnp.float32)
        for kh in range(3):
            patch = buf_ref[pl.ds(kh, H), :, :].reshape(H * W, 3 * c)
            acc = acc + jnp.dot(patch, w2_ref[kh],
                                preferred_element_type=jnp.float32)
        y2_ref[0] = acc.astype(jnp.bfloat16)
        stat_ref[0] = _stat_rows(acc, c)
    return body


def _make_stats3_kernel(M, eps, pk):
    def body(y2_ref, st2_ref, g2_ref, b2_ref, sum_ref, gram_ref):
        cpk = y2_ref.shape[-1]
        sc, sh = _affine_from_stats(st2_ref[...], g2_ref[...], b2_ref[...],
                                    M, eps)
        ap = jnp.maximum(
            y2_ref[0].astype(jnp.float32) * _dup(sc, pk) + _dup(sh, pk), 0.0)
        abp = ap.astype(jnp.bfloat16)
        s = jnp.sum(abp.astype(jnp.float32), axis=0, keepdims=True)
        row = lax.broadcasted_iota(jnp.int32, (8, cpk), 0)
        sum_ref[0] = jnp.where(row == 0, s, 0.0)
        gram_ref[0] = lax.dot_general(abp, abp, (((0,), (0,)), ((), ())),
                                      preferred_element_type=jnp.float32)
    return body


def _make_final_kernel(M, eps, pk):
    def body(y2_ref, st2_ref, g2_ref, b2_ref, w3_ref, sum3_ref, gram_ref,
             g3_ref, b3_ref, x_ref, o_ref):
        sc2, sh2 = _affine_from_stats(st2_ref[...], g2_ref[...], b2_ref[...],
                                      M, eps)
        a = jnp.maximum(y2_ref[...].astype(jnp.float32) * sc2 + sh2, 0.0)
        ab = a.astype(jnp.bfloat16)
        w3b = w3_ref[...].astype(jnp.bfloat16)
        y3 = jnp.dot(ab, w3b, preferred_element_type=jnp.float32)

        w3f = w3b.astype(jnp.float32)
        c = y2_ref.shape[-1]
        sp = jnp.sum(sum3_ref[:, 0, :], axis=0, keepdims=True)
        srow = _fold(sp, pk, c)
        s3 = jnp.dot(srow, w3f, preferred_element_type=jnp.float32)
        gp = jnp.sum(gram_ref[...], axis=0)
        g = gp[0:c, 0:c]
        for j in range(1, pk):
            g = g + gp[j * c:(j + 1) * c, j * c:(j + 1) * c]
        t = jnp.dot(g, w3f, preferred_element_type=jnp.float32)
        q3 = jnp.sum(t * w3f, axis=0, keepdims=True)
        mean = s3 * (1.0 / M)
        var = jnp.maximum(q3 * (1.0 / M) - mean * mean, 0.0)
        scale3 = g3_ref[...] * lax.rsqrt(var + eps)
        shift3 = b3_ref[...] - mean * scale3

        o_ref[...] = jnp.maximum(y3 * scale3 + shift3 + x_ref[...], 0.0)
    return body


def kernel(x, w1, b1, w2, b2, w3, b3, g1, be1, g2, be2, g3, be3):
    N, C, H, W = x.shape
    oc = w1.shape[0]
    HW = H * W
    M = N * HW
    eps = 1e-5
    pk = 4 if HW % 4 == 0 else (2 if HW % 2 == 0 else 1)
    tm = _pick_tm(M)
    while tm % pk:
        pk //= 2
    nt = M // tm
    cpk = oc * pk
    cp = pltpu.CompilerParams(dimension_semantics=("parallel",),
                              vmem_limit_bytes=_VMEM_LIMIT)

    x2d = jnp.transpose(x, (0, 2, 3, 1)).reshape(M, C)
    w1t = w1.reshape(oc, C).T
    w1bd = jnp.kron(jnp.eye(pk, dtype=w1t.dtype), w1t)
    xq = x2d.reshape(M // pk, pk * C)
    w2b = jnp.transpose(w2, (2, 3, 1, 0)).reshape(3, 3 * oc, oc).astype(
        jnp.bfloat16)
    w3t = w3.reshape(C, oc).T
    g1r, b1r = g1.reshape(1, oc), be1.reshape(1, oc)
    g2r, b2r = g2.reshape(1, oc), be2.reshape(1, oc)
    g3r, b3r = g3.reshape(1, C), be3.reshape(1, C)

    y1q, st1 = pl.pallas_call(
        _conv1_kernel,
        out_shape=(jax.ShapeDtypeStruct((M // pk, cpk), jnp.bfloat16),
                   jax.ShapeDtypeStruct((nt, 8, cpk), jnp.float32)),
        grid=(nt,),
        in_specs=[pl.BlockSpec((tm // pk, pk * C), lambda i: (i, 0)),
                  pl.BlockSpec((pk * C, cpk), lambda i: (0, 0))],
        out_specs=(pl.BlockSpec((tm // pk, cpk), lambda i: (i, 0)),
                   pl.BlockSpec((1, 8, cpk), lambda i: (i, 0, 0))),
        compiler_params=cp,
    )(xq, w1bd)
    y1 = y1q.reshape(M, oc)

    y1n = y1.reshape(N, HW, oc)
    y2, st2 = pl.pallas_call(
        _make_conv3_kernel(H, W, M, eps, pk),
        out_shape=(jax.ShapeDtypeStruct((N, HW, oc), jnp.bfloat16),
                   jax.ShapeDtypeStruct((N, 8, oc), jnp.float32)),
        grid=(N,),
        in_specs=[pl.BlockSpec((1, HW, oc), lambda i: (i, 0, 0)),
                  pl.BlockSpec((nt, 8, cpk), lambda i: (0, 0, 0)),
                  pl.BlockSpec((1, oc), lambda i: (0, 0)),
                  pl.BlockSpec((1, oc), lambda i: (0, 0)),
                  pl.BlockSpec((3, 3 * oc, oc), lambda i: (0, 0, 0))],
        out_specs=(pl.BlockSpec((1, HW, oc), lambda i: (i, 0, 0)),
                   pl.BlockSpec((1, 8, oc), lambda i: (i, 0, 0))),
        scratch_shapes=[pltpu.VMEM((H + 2, W + 2, oc), jnp.bfloat16),
                        pltpu.VMEM((H + 2, W, 3 * oc), jnp.bfloat16)],
        compiler_params=cp,
    )(y1n, st1, g1r, b1r, w2b)

    y2q = y2.reshape(N, HW // pk, cpk)
    st3, gram = pl.pallas_call(
        _make_stats3_kernel(M, eps, pk),
        out_shape=(jax.ShapeDtypeStruct((N, 8, cpk), jnp.float32),
                   jax.ShapeDtypeStruct((N, cpk, cpk), jnp.float32)),
        grid=(N,),
        in_specs=[pl.BlockSpec((1, HW // pk, cpk), lambda i: (i, 0, 0)),
                  pl.BlockSpec((N, 8, oc), lambda i: (0, 0, 0)),
                  pl.BlockSpec((1, oc), lambda i: (0, 0)),
                  pl.BlockSpec((1, oc), lambda i: (0, 0))],
        out_specs=(pl.BlockSpec((1, 8, cpk), lambda i: (i, 0, 0)),
                   pl.BlockSpec((1, cpk, cpk), lambda i: (i, 0, 0))),
        compiler_params=cp,
    )(y2q, st2, g2r, b2r)

    y2f = y2.reshape(M, oc)
    out2d = pl.pallas_call(
        _make_final_kernel(M, eps, pk),
        out_shape=jax.ShapeDtypeStruct((M, C), jnp.float32),
        grid=(nt,),
        in_specs=[pl.BlockSpec((tm, oc), lambda i: (i, 0)),
                  pl.BlockSpec((N, 8, oc), lambda i: (0, 0, 0)),
                  pl.BlockSpec((1, oc), lambda i: (0, 0)),
                  pl.BlockSpec((1, oc), lambda i: (0, 0)),
                  pl.BlockSpec((oc, C), lambda i: (0, 0)),
                  pl.BlockSpec((N, 8, cpk), lambda i: (0, 0, 0)),
                  pl.BlockSpec((N, cpk, cpk), lambda i: (0, 0, 0)),
                  pl.BlockSpec((1, C), lambda i: (0, 0)),
                  pl.BlockSpec((1, C), lambda i: (0, 0)),
                  pl.BlockSpec((tm, C), lambda i: (i, 0))],
        out_specs=pl.BlockSpec((tm, C), lambda i: (i, 0)),
        compiler_params=cp,
    )(y2f, st2, g2r, b2r, w3t, st3, gram, g3r, b3r, x2d)

    out = out2d.reshape(N, H, W, C)
    return jnp.transpose(out, (0, 3, 1, 2))


# speedup vs baseline: 1.5873x; 1.5873x over previous
import jax
import jax.numpy as jnp
from jax import lax
from jax.experimental import pallas as pl
from jax.experimental.pallas import tpu as pltpu

_VMEM_LIMIT = 96 * 1024 * 1024


def _pick_tm(m, cap=6400):
    start = min(cap, m)
    start -= start % 8
    for tm in range(start, 7, -8):
        if m % tm == 0:
            return tm
    return m


def _stat_rows(y, ch):
    s = jnp.sum(y, axis=0, keepdims=True)
    q = jnp.sum(y * y, axis=0, keepdims=True)
    row = lax.broadcasted_iota(jnp.int32, (8, ch), 0)
    return jnp.where(row == 0, s, jnp.where(row == 1, q, 0.0))


def _affine_from_stats(st, gamma, beta, count, eps):
    s = jnp.sum(st[:, 0, :], axis=0, keepdims=True)
    q = jnp.sum(st[:, 1, :], axis=0, keepdims=True)
    mean = s * (1.0 / count)
    var = jnp.maximum(q * (1.0 / count) - mean * mean, 0.0)
    scale = gamma * lax.rsqrt(var + eps)
    shift = beta - mean * scale
    return scale, shift


def _conv1_kernel(x_ref, w_ref, y_ref, stat_ref):
    xb = x_ref[...].astype(jnp.bfloat16)
    wb = w_ref[...].astype(jnp.bfloat16)
    y = jnp.dot(xb, wb, preferred_element_type=jnp.float32)
    y_ref[...] = y.astype(jnp.bfloat16)
    stat_ref[0] = _stat_rows(y, y.shape[1])


def _make_conv3_kernel(H, W, M, eps):
    def body(y1_ref, st1_ref, g1_ref, b1_ref, w2_ref, y2_ref, stat_ref,
             pad_ref, buf_ref):
        c = y1_ref.shape[-1]
        sc, sh = _affine_from_stats(st1_ref[...], g1_ref[...], b1_ref[...],
                                    M, eps)
        a = jnp.maximum(y1_ref[0].astype(jnp.float32) * sc + sh, 0.0)
        zc = jnp.zeros((W + 2, c), jnp.bfloat16)
        pad_ref[0] = zc
        pad_ref[H + 1] = zc
        zr = jnp.zeros((H + 2, c), jnp.bfloat16)
        pad_ref[:, 0, :] = zr
        pad_ref[:, W + 1, :] = zr
        pad_ref[pl.ds(1, H), pl.ds(1, W), :] = (
            a.reshape(H, W, c).astype(jnp.bfloat16))

        for kw in range(3):
            buf_ref[:, :, pl.ds(kw * c, c)] = pad_ref[:, pl.ds(kw, W), :]

        acc = jnp.zeros((H * W, c), jnp.float32)
        for kh in range(3):
            patch = buf_ref[pl.ds(kh, H), :, :].reshape(H * W, 3 * c)
            acc = acc + jnp.dot(patch, w2_ref[kh],
                                preferred_element_type=jnp.float32)
        y2_ref[0] = acc.astype(jnp.bfloat16)
        stat_ref[0] = _stat_rows(acc, c)
    return body


def _make_stats3_kernel(M, eps):
    def body(y2_ref, st2_ref, g2_ref, b2_ref, sum_ref, gram_ref):
        c = y2_ref.shape[-1]
        sc, sh = _affine_from_stats(st2_ref[...], g2_ref[...], b2_ref[...],
                                    M, eps)
        a = jnp.maximum(y2_ref[0].astype(jnp.float32) * sc + sh, 0.0)
        ab = a.astype(jnp.bfloat16)
        af = ab.astype(jnp.float32)
        s = jnp.sum(af, axis=0, keepdims=True)
        row = lax.broadcasted_iota(jnp.int32, (8, c), 0)
        sum_ref[0] = jnp.where(row == 0, s, 0.0)
        gram_ref[0] = lax.dot_general(ab, ab, (((0,), (0,)), ((), ())),
                                      preferred_element_type=jnp.float32)
    return body


def _make_final_kernel(M, eps):
    def body(y2_ref, st2_ref, g2_ref, b2_ref, w3_ref, sum3_ref, gram_ref,
             g3_ref, b3_ref, x_ref, o_ref):
        sc2, sh2 = _affine_from_stats(st2_ref[...], g2_ref[...], b2_ref[...],
                                      M, eps)
        a = jnp.maximum(y2_ref[...].astype(jnp.float32) * sc2 + sh2, 0.0)
        ab = a.astype(jnp.bfloat16)
        w3b = w3_ref[...].astype(jnp.bfloat16)
        y3 = jnp.dot(ab, w3b, preferred_element_type=jnp.float32)

        w3f = w3b.astype(jnp.float32)
        srow = jnp.sum(sum3_ref[:, 0, :], axis=0, keepdims=True)
        s3 = jnp.dot(srow, w3f, preferred_element_type=jnp.float32)
        g = jnp.sum(gram_ref[...], axis=0)
        t = jnp.dot(g, w3f, preferred_element_type=jnp.float32)
        q3 = jnp.sum(t * w3f, axis=0, keepdims=True)
        mean = s3 * (1.0 / M)
        var = jnp.maximum(q3 * (1.0 / M) - mean * mean, 0.0)
        scale3 = g3_ref[...] * lax.rsqrt(var + eps)
        shift3 = b3_ref[...] - mean * scale3

        o_ref[...] = jnp.maximum(y3 * scale3 + shift3 + x_ref[...], 0.0)
    return body


def kernel(x, w1, b1, w2, b2, w3, b3, g1, be1, g2, be2, g3, be3):
    N, C, H, W = x.shape
    oc = w1.shape[0]
    HW = H * W
    M = N * HW
    eps = 1e-5
    tm = _pick_tm(M)
    nt = M // tm
    cp = pltpu.CompilerParams(dimension_semantics=("parallel",),
                              vmem_limit_bytes=_VMEM_LIMIT)

    x2d = jnp.transpose(x, (0, 2, 3, 1)).reshape(M, C)
    ocp = ((oc + 127) // 128) * 128
    po = ocp - oc
    w1t = jnp.pad(w1.reshape(oc, C).T, ((0, 0), (0, po)))
    w2b = jnp.pad(jnp.transpose(w2, (2, 3, 1, 0)),
                  ((0, 0), (0, 0), (0, po), (0, po))).reshape(
        3, 3 * ocp, ocp).astype(jnp.bfloat16)
    w3t = jnp.pad(w3.reshape(C, oc).T, ((0, po), (0, 0)))
    g1r, b1r = (jnp.pad(g1, (0, po)).reshape(1, ocp),
                jnp.pad(be1, (0, po)).reshape(1, ocp))
    g2r, b2r = (jnp.pad(g2, (0, po)).reshape(1, ocp),
                jnp.pad(be2, (0, po)).reshape(1, ocp))
    g3r, b3r = g3.reshape(1, C), be3.reshape(1, C)
    oc = ocp

    y1, st1 = pl.pallas_call(
        _conv1_kernel,
        out_shape=(jax.ShapeDtypeStruct((M, oc), jnp.bfloat16),
                   jax.ShapeDtypeStruct((nt, 8, oc), jnp.float32)),
        grid=(nt,),
        in_specs=[pl.BlockSpec((tm, C), lambda i: (i, 0)),
                  pl.BlockSpec((C, oc), lambda i: (0, 0))],
        out_specs=(pl.BlockSpec((tm, oc), lambda i: (i, 0)),
                   pl.BlockSpec((1, 8, oc), lambda i: (i, 0, 0))),
        compiler_params=cp,
    )(x2d, w1t)

    y1n = y1.reshape(N, HW, oc)
    y2, st2 = pl.pallas_call(
        _make_conv3_kernel(H, W, M, eps),
        out_shape=(jax.ShapeDtypeStruct((N, HW, oc), jnp.bfloat16),
                   jax.ShapeDtypeStruct((N, 8, oc), jnp.float32)),
        grid=(N,),
        in_specs=[pl.BlockSpec((1, HW, oc), lambda i: (i, 0, 0)),
                  pl.BlockSpec((nt, 8, oc), lambda i: (0, 0, 0)),
                  pl.BlockSpec((1, oc), lambda i: (0, 0)),
                  pl.BlockSpec((1, oc), lambda i: (0, 0)),
                  pl.BlockSpec((3, 3 * oc, oc), lambda i: (0, 0, 0))],
        out_specs=(pl.BlockSpec((1, HW, oc), lambda i: (i, 0, 0)),
                   pl.BlockSpec((1, 8, oc), lambda i: (i, 0, 0))),
        scratch_shapes=[pltpu.VMEM((H + 2, W + 2, oc), jnp.bfloat16),
                        pltpu.VMEM((H + 2, W, 3 * oc), jnp.bfloat16)],
        compiler_params=cp,
    )(y1n, st1, g1r, b1r, w2b)

    st3, gram = pl.pallas_call(
        _make_stats3_kernel(M, eps),
        out_shape=(jax.ShapeDtypeStruct((N, 8, oc), jnp.float32),
                   jax.ShapeDtypeStruct((N, oc, oc), jnp.float32)),
        grid=(N,),
        in_specs=[pl.BlockSpec((1, HW, oc), lambda i: (i, 0, 0)),
                  pl.BlockSpec((N, 8, oc), lambda i: (0, 0, 0)),
                  pl.BlockSpec((1, oc), lambda i: (0, 0)),
                  pl.BlockSpec((1, oc), lambda i: (0, 0))],
        out_specs=(pl.BlockSpec((1, 8, oc), lambda i: (i, 0, 0)),
                   pl.BlockSpec((1, oc, oc), lambda i: (i, 0, 0))),
        compiler_params=cp,
    )(y2, st2, g2r, b2r)

    y2f = y2.reshape(M, oc)
    out2d = pl.pallas_call(
        _make_final_kernel(M, eps),
        out_shape=jax.ShapeDtypeStruct((M, C), jnp.float32),
        grid=(nt,),
        in_specs=[pl.BlockSpec((tm, oc), lambda i: (i, 0)),
                  pl.BlockSpec((N, 8, oc), lambda i: (0, 0, 0)),
                  pl.BlockSpec((1, oc), lambda i: (0, 0)),
                  pl.BlockSpec((1, oc), lambda i: (0, 0)),
                  pl.BlockSpec((oc, C), lambda i: (0, 0)),
                  pl.BlockSpec((N, 8, oc), lambda i: (0, 0, 0)),
                  pl.BlockSpec((N, oc, oc), lambda i: (0, 0, 0)),
                  pl.BlockSpec((1, C), lambda i: (0, 0)),
                  pl.BlockSpec((1, C), lambda i: (0, 0)),
                  pl.BlockSpec((tm, C), lambda i: (i, 0))],
        out_specs=pl.BlockSpec((tm, C), lambda i: (i, 0)),
        compiler_params=cp,
    )(y2f, st2, g2r, b2r, w3t, st3, gram, g3r, b3r, x2d)

    out = out2d.reshape(N, H, W, C)
    return jnp.transpose(out, (0, 3, 1, 2))


# speedup vs baseline: 1.7891x; 1.1271x over previous
import jax
import jax.numpy as jnp
from jax import lax
from jax.experimental import pallas as pl
from jax.experimental.pallas import tpu as pltpu

_VMEM_LIMIT = 96 * 1024 * 1024


def _pick_tm(m, cap=6400):
    start = min(cap, m)
    start -= start % 8
    for tm in range(start, 7, -8):
        if m % tm == 0:
            return tm
    return m


def _stat_rows(y, ch):
    s = jnp.sum(y, axis=0, keepdims=True)
    q = jnp.sum(y * y, axis=0, keepdims=True)
    row = lax.broadcasted_iota(jnp.int32, (8, ch), 0)
    return jnp.where(row == 0, s, jnp.where(row == 1, q, 0.0))


def _affine_from_stats(st, gamma, beta, count, eps):
    s = jnp.sum(st[:, 0, :], axis=0, keepdims=True)
    q = jnp.sum(st[:, 1, :], axis=0, keepdims=True)
    mean = s * (1.0 / count)
    var = jnp.maximum(q * (1.0 / count) - mean * mean, 0.0)
    scale = gamma * lax.rsqrt(var + eps)
    shift = beta - mean * scale
    return scale, shift


def _conv1_kernel(x_ref, w_ref, y_ref, stat_ref):
    xb = x_ref[...].astype(jnp.bfloat16)
    wb = w_ref[...].astype(jnp.bfloat16)
    y = jnp.dot(xb, wb, preferred_element_type=jnp.float32)
    y_ref[...] = y.astype(jnp.bfloat16)
    stat_ref[0] = _stat_rows(y, y.shape[1])


def _make_conv3_kernel(H, W, M, eps):
    def body(y1_ref, st1_ref, g1_ref, b1_ref, w2_ref, y2_ref, stat_ref,
             buf_ref):
        c = y1_ref.shape[-1]
        sc, sh = _affine_from_stats(st1_ref[...], g1_ref[...], b1_ref[...],
                                    M, eps)
        a = jnp.maximum(y1_ref[0].astype(jnp.float32) * sc + sh, 0.0)
        ab = a.reshape(H, W, c).astype(jnp.bfloat16)
        zrow = jnp.zeros((W, 3 * c), jnp.bfloat16)
        buf_ref[0] = zrow
        buf_ref[H + 1] = zrow
        zc = jnp.zeros((H, 1, c), jnp.bfloat16)
        buf_ref[pl.ds(1, H), pl.ds(0, 1), pl.ds(0, c)] = zc
        buf_ref[pl.ds(1, H), pl.ds(W - 1, 1), pl.ds(2 * c, c)] = zc
        buf_ref[pl.ds(1, H), pl.ds(1, W - 1), pl.ds(0, c)] = ab[:, :W - 1, :]
        buf_ref[pl.ds(1, H), :, pl.ds(c, c)] = ab
        buf_ref[pl.ds(1, H), pl.ds(0, W - 1), pl.ds(2 * c, c)] = ab[:, 1:, :]

        acc = jnp.zeros((H * W, c), jnp.float32)
        for kh in range(3):
            patch = buf_ref[pl.ds(kh, H), :, :].reshape(H * W, 3 * c)
            acc = acc + jnp.dot(patch, w2_ref[kh],
                                preferred_element_type=jnp.float32)
        y2_ref[0] = acc.astype(jnp.bfloat16)
        stat_ref[0] = _stat_rows(acc, c)
    return body


def _make_stats3_kernel(M, eps):
    def body(y2_ref, st2_ref, g2_ref, b2_ref, sum_ref, gram_ref):
        c = y2_ref.shape[-1]
        sc, sh = _affine_from_stats(st2_ref[...], g2_ref[...], b2_ref[...],
                                    M, eps)
        a = jnp.maximum(y2_ref[0].astype(jnp.float32) * sc + sh, 0.0)
        ab = a.astype(jnp.bfloat16)
        af = ab.astype(jnp.float32)
        s = jnp.sum(af, axis=0, keepdims=True)
        row = lax.broadcasted_iota(jnp.int32, (8, c), 0)
        sum_ref[0] = jnp.where(row == 0, s, 0.0)
        gram_ref[0] = lax.dot_general(ab, ab, (((0,), (0,)), ((), ())),
                                      preferred_element_type=jnp.float32)
    return body


def _make_final_kernel(M, eps):
    def body(y2_ref, st2_ref, g2_ref, b2_ref, w3_ref, sum3_ref, gram_ref,
             g3_ref, b3_ref, x_ref, o_ref):
        sc2, sh2 = _affine_from_stats(st2_ref[...], g2_ref[...], b2_ref[...],
                                      M, eps)
        a = jnp.maximum(y2_ref[...].astype(jnp.float32) * sc2 + sh2, 0.0)
        ab = a.astype(jnp.bfloat16)
        w3b = w3_ref[...].astype(jnp.bfloat16)
        y3 = jnp.dot(ab, w3b, preferred_element_type=jnp.float32)

        w3f = w3b.astype(jnp.float32)
        srow = jnp.sum(sum3_ref[:, 0, :], axis=0, keepdims=True)
        s3 = jnp.dot(srow, w3f, preferred_element_type=jnp.float32)
        g = jnp.sum(gram_ref[...], axis=0)
        t = jnp.dot(g, w3f, preferred_element_type=jnp.float32)
        q3 = jnp.sum(t * w3f, axis=0, keepdims=True)
        mean = s3 * (1.0 / M)
        var = jnp.maximum(q3 * (1.0 / M) - mean * mean, 0.0)
        scale3 = g3_ref[...] * lax.rsqrt(var + eps)
        shift3 = b3_ref[...] - mean * scale3

        o_ref[...] = jnp.maximum(y3 * scale3 + shift3 + x_ref[...], 0.0)
    return body


def kernel(x, w1, b1, w2, b2, w3, b3, g1, be1, g2, be2, g3, be3):
    N, C, H, W = x.shape
    oc = w1.shape[0]
    HW = H * W
    M = N * HW
    eps = 1e-5
    tm = _pick_tm(M)
    nt = M // tm
    cp = pltpu.CompilerParams(dimension_semantics=("parallel",),
                              vmem_limit_bytes=_VMEM_LIMIT)

    x2d = jnp.transpose(x, (0, 2, 3, 1)).reshape(M, C)
    w1t = w1.reshape(oc, C).T
    w2b = jnp.transpose(w2, (2, 3, 1, 0)).reshape(3, 3 * oc, oc).astype(
        jnp.bfloat16)
    w3t = w3.reshape(C, oc).T
    g1r, b1r = g1.reshape(1, oc), be1.reshape(1, oc)
    g2r, b2r = g2.reshape(1, oc), be2.reshape(1, oc)
    g3r, b3r = g3.reshape(1, C), be3.reshape(1, C)

    y1, st1 = pl.pallas_call(
        _conv1_kernel,
        out_shape=(jax.ShapeDtypeStruct((M, oc), jnp.bfloat16),
                   jax.ShapeDtypeStruct((nt, 8, oc), jnp.float32)),
        grid=(nt,),
        in_specs=[pl.BlockSpec((tm, C), lambda i: (i, 0)),
                  pl.BlockSpec((C, oc), lambda i: (0, 0))],
        out_specs=(pl.BlockSpec((tm, oc), lambda i: (i, 0)),
                   pl.BlockSpec((1, 8, oc), lambda i: (i, 0, 0))),
        compiler_params=cp,
    )(x2d, w1t)

    y1n = y1.reshape(N, HW, oc)
    y2, st2 = pl.pallas_call(
        _make_conv3_kernel(H, W, M, eps),
        out_shape=(jax.ShapeDtypeStruct((N, HW, oc), jnp.bfloat16),
                   jax.ShapeDtypeStruct((N, 8, oc), jnp.float32)),
        grid=(N,),
        in_specs=[pl.BlockSpec((1, HW, oc), lambda i: (i, 0, 0)),
                  pl.BlockSpec((nt, 8, oc), lambda i: (0, 0, 0)),
                  pl.BlockSpec((1, oc), lambda i: (0, 0)),
                  pl.BlockSpec((1, oc), lambda i: (0, 0)),
                  pl.BlockSpec((3, 3 * oc, oc), lambda i: (0, 0, 0))],
        out_specs=(pl.BlockSpec((1, HW, oc), lambda i: (i, 0, 0)),
                   pl.BlockSpec((1, 8, oc), lambda i: (i, 0, 0))),
        scratch_shapes=[pltpu.VMEM((H + 2, W, 3 * oc), jnp.bfloat16)],
        compiler_params=cp,
    )(y1n, st1, g1r, b1r, w2b)

    st3, gram = pl.pallas_call(
        _make_stats3_kernel(M, eps),
        out_shape=(jax.ShapeDtypeStruct((N, 8, oc), jnp.float32),
                   jax.ShapeDtypeStruct((N, oc, oc), jnp.float32)),
        grid=(N,),
        in_specs=[pl.BlockSpec((1, HW, oc), lambda i: (i, 0, 0)),
                  pl.BlockSpec((N, 8, oc), lambda i: (0, 0, 0)),
                  pl.BlockSpec((1, oc), lambda i: (0, 0)),
                  pl.BlockSpec((1, oc), lambda i: (0, 0))],
        out_specs=(pl.BlockSpec((1, 8, oc), lambda i: (i, 0, 0)),
                   pl.BlockSpec((1, oc, oc), lambda i: (i, 0, 0))),
        compiler_params=cp,
    )(y2, st2, g2r, b2r)

    y2f = y2.reshape(M, oc)
    out2d = pl.pallas_call(
        _make_final_kernel(M, eps),
        out_shape=jax.ShapeDtypeStruct((M, C), jnp.float32),
        grid=(nt,),
        in_specs=[pl.BlockSpec((tm, oc), lambda i: (i, 0)),
                  pl.BlockSpec((N, 8, oc), lambda i: (0, 0, 0)),
                  pl.BlockSpec((1, oc), lambda i: (0, 0)),
                  pl.BlockSpec((1, oc), lambda i: (0, 0)),
                  pl.BlockSpec((oc, C), lambda i: (0, 0)),
                  pl.BlockSpec((N, 8, oc), lambda i: (0, 0, 0)),
                  pl.BlockSpec((N, oc, oc), lambda i: (0, 0, 0)),
                  pl.BlockSpec((1, C), lambda i: (0, 0)),
                  pl.BlockSpec((1, C), lambda i: (0, 0)),
                  pl.BlockSpec((tm, C), lambda i: (i, 0))],
        out_specs=pl.BlockSpec((tm, C), lambda i: (i, 0)),
        compiler_params=cp,
    )(y2f, st2, g2r, b2r, w3t, st3, gram, g3r, b3r, x2d)

    out = out2d.reshape(N, H, W, C)
    return jnp.transpose(out, (0, 3, 1, 2))


# speedup vs baseline: 1.8624x; 1.0409x over previous
import jax
import jax.numpy as jnp
from jax import lax
from jax.experimental import pallas as pl
from jax.experimental.pallas import tpu as pltpu

_VMEM_LIMIT = 96 * 1024 * 1024


def _pick_tm(m, cap=6400):
    start = min(cap, m)
    start -= start % 8
    for tm in range(start, 7, -8):
        if m % tm == 0:
            return tm
    return m


def _stat_rows(y, ch):
    s = jnp.sum(y, axis=0, keepdims=True)
    q = jnp.sum(y * y, axis=0, keepdims=True)
    row = lax.broadcasted_iota(jnp.int32, (8, ch), 0)
    return jnp.where(row == 0, s, jnp.where(row == 1, q, 0.0))


def _affine_from_stats(st, gamma, beta, count, eps):
    s = jnp.sum(st[:, 0, :], axis=0, keepdims=True)
    q = jnp.sum(st[:, 1, :], axis=0, keepdims=True)
    mean = s * (1.0 / count)
    var = jnp.maximum(q * (1.0 / count) - mean * mean, 0.0)
    scale = gamma * lax.rsqrt(var + eps)
    shift = beta - mean * scale
    return scale, shift


def _conv1_kernel(x_ref, w_ref, y_ref, stat_ref):
    xb = x_ref[...].astype(jnp.bfloat16)
    wb = w_ref[...].astype(jnp.bfloat16)
    y = jnp.dot(xb, wb, preferred_element_type=jnp.float32)
    y_ref[...] = y.astype(jnp.bfloat16)
    stat_ref[0] = _stat_rows(y, y.shape[1])


def _make_conv3_kernel(H, W, M, eps):
    def body(y1_ref, st1_ref, g1_ref, b1_ref, w2_ref, y2_ref, stat_ref,
             buf_ref):
        c = y1_ref.shape[-1]
        sc, sh = _affine_from_stats(st1_ref[...], g1_ref[...], b1_ref[...],
                                    M, eps)
        a = jnp.maximum(y1_ref[0].astype(jnp.float32) * sc + sh, 0.0)
        ab = a.reshape(H, W, c).astype(jnp.bfloat16)
        zrow = jnp.zeros((W, 3 * c), jnp.bfloat16)
        buf_ref[0] = zrow
        buf_ref[H + 1] = zrow
        zc = jnp.zeros((H, 1, c), jnp.bfloat16)
        buf_ref[pl.ds(1, H), pl.ds(0, 1), pl.ds(0, c)] = zc
        buf_ref[pl.ds(1, H), pl.ds(W - 1, 1), pl.ds(2 * c, c)] = zc
        buf_ref[pl.ds(1, H), pl.ds(1, W - 1), pl.ds(0, c)] = ab[:, :W - 1, :]
        buf_ref[pl.ds(1, H), :, pl.ds(c, c)] = ab
        buf_ref[pl.ds(1, H), pl.ds(0, W - 1), pl.ds(2 * c, c)] = ab[:, 1:, :]

        acc = jnp.zeros((H * W, c), jnp.float32)
        for kh in range(3):
            patch = buf_ref[pl.ds(kh, H), :, :].reshape(H * W, 3 * c)
            acc = acc + jnp.dot(patch, w2_ref[kh],
                                preferred_element_type=jnp.float32)
        y2_ref[0] = acc.astype(jnp.bfloat16)
        stat_ref[0] = _stat_rows(acc, c)
    return body


def _make_stats3_kernel(M, eps):
    def body(y2_ref, st2_ref, g2_ref, b2_ref, sum_ref, gram_ref):
        c = y2_ref.shape[-1]
        sc, sh = _affine_from_stats(st2_ref[...], g2_ref[...], b2_ref[...],
                                    M, eps)
        a = jnp.maximum(y2_ref[0].astype(jnp.float32) * sc + sh, 0.0)
        ab = a.astype(jnp.bfloat16)
        af = ab.astype(jnp.float32)
        s = jnp.sum(af, axis=0, keepdims=True)
        row = lax.broadcasted_iota(jnp.int32, (8, c), 0)
        sum_ref[0] = jnp.where(row == 0, s, 0.0)
        gram_ref[0] = lax.dot_general(ab, ab, (((0,), (0,)), ((), ())),
                                      preferred_element_type=jnp.float32)
    return body


def _make_final_kernel(M, eps):
    def body(y2_ref, st2_ref, g2_ref, b2_ref, w3_ref, sum3_ref, gram_ref,
             g3_ref, b3_ref, x_ref, o_ref):
        sc2, sh2 = _affine_from_stats(st2_ref[...], g2_ref[...], b2_ref[...],
                                      M, eps)
        a = jnp.maximum(y2_ref[...].astype(jnp.float32) * sc2 + sh2, 0.0)
        ab = a.astype(jnp.bfloat16)
        w3b = w3_ref[...].astype(jnp.bfloat16)
        y3 = jnp.dot(ab, w3b, preferred_element_type=jnp.float32)

        w3f = w3b.astype(jnp.float32)
        srow = jnp.sum(sum3_ref[:, 0, :], axis=0, keepdims=True)
        s3 = jnp.dot(srow, w3f, preferred_element_type=jnp.float32)
        g = jnp.sum(gram_ref[...], axis=0)
        t = jnp.dot(g, w3f, preferred_element_type=jnp.float32)
        q3 = jnp.sum(t * w3f, axis=0, keepdims=True)
        mean = s3 * (1.0 / M)
        var = jnp.maximum(q3 * (1.0 / M) - mean * mean, 0.0)
        scale3 = g3_ref[...] * lax.rsqrt(var + eps)
        shift3 = b3_ref[...] - mean * scale3

        o_ref[...] = jnp.maximum(y3 * scale3 + shift3 + x_ref[...], 0.0)
    return body


def kernel(x, w1, b1, w2, b2, w3, b3, g1, be1, g2, be2, g3, be3):
    N, C, H, W = x.shape
    oc = w1.shape[0]
    HW = H * W
    M = N * HW
    eps = 1e-5
    tm = _pick_tm(M)
    nt = M // tm
    cp = pltpu.CompilerParams(dimension_semantics=("parallel",),
                              vmem_limit_bytes=_VMEM_LIMIT)

    x2d = jnp.transpose(x, (0, 2, 3, 1)).reshape(M, C)
    w1t = w1.reshape(oc, C).T
    w2b = jnp.transpose(w2, (2, 3, 1, 0)).reshape(3, 3 * oc, oc).astype(
        jnp.bfloat16)
    w3t = w3.reshape(C, oc).T
    g1r, b1r = g1.reshape(1, oc), be1.reshape(1, oc)
    g2r, b2r = g2.reshape(1, oc), be2.reshape(1, oc)
    g3r, b3r = g3.reshape(1, C), be3.reshape(1, C)

    y1, st1 = pl.pallas_call(
        _conv1_kernel,
        out_shape=(jax.ShapeDtypeStruct((M, oc), jnp.bfloat16),
                   jax.ShapeDtypeStruct((nt, 8, oc), jnp.float32)),
        grid=(nt,),
        in_specs=[pl.BlockSpec((tm, C), lambda i: (i, 0)),
                  pl.BlockSpec((C, oc), lambda i: (0, 0))],
        out_specs=(pl.BlockSpec((tm, oc), lambda i: (i, 0)),
                   pl.BlockSpec((1, 8, oc), lambda i: (i, 0, 0))),
        compiler_params=cp,
    )(x2d, w1t)

    y1n = y1.reshape(N, HW, oc)
    y2, st2 = pl.pallas_call(
        _make_conv3_kernel(H, W, M, eps),
        out_shape=(jax.ShapeDtypeStruct((N, HW, oc), jnp.bfloat16),
                   jax.ShapeDtypeStruct((N, 8, oc), jnp.float32)),
        grid=(N,),
        in_specs=[pl.BlockSpec((1, HW, oc), lambda i: (i, 0, 0)),
                  pl.BlockSpec((nt, 8, oc), lambda i: (0, 0, 0)),
                  pl.BlockSpec((1, oc), lambda i: (0, 0)),
                  pl.BlockSpec((1, oc), lambda i: (0, 0)),
                  pl.BlockSpec((3, 3 * oc, oc), lambda i: (0, 0, 0))],
        out_specs=(pl.BlockSpec((1, HW, oc), lambda i: (i, 0, 0)),
                   pl.BlockSpec((1, 8, oc), lambda i: (i, 0, 0))),
        scratch_shapes=[pltpu.VMEM((H + 2, W, 3 * oc), jnp.bfloat16)],
        compiler_params=cp,
    )(y1n, st1, g1r, b1r, w2b)

    y2m = y2.reshape(nt, tm, oc)
    st3, gram = pl.pallas_call(
        _make_stats3_kernel(M, eps),
        out_shape=(jax.ShapeDtypeStruct((nt, 8, oc), jnp.float32),
                   jax.ShapeDtypeStruct((nt, oc, oc), jnp.float32)),
        grid=(nt,),
        in_specs=[pl.BlockSpec((1, tm, oc), lambda i: (i, 0, 0)),
                  pl.BlockSpec((N, 8, oc), lambda i: (0, 0, 0)),
                  pl.BlockSpec((1, oc), lambda i: (0, 0)),
                  pl.BlockSpec((1, oc), lambda i: (0, 0))],
        out_specs=(pl.BlockSpec((1, 8, oc), lambda i: (i, 0, 0)),
                   pl.BlockSpec((1, oc, oc), lambda i: (i, 0, 0))),
        compiler_params=cp,
    )(y2m, st2, g2r, b2r)

    y2f = y2.reshape(M, oc)
    out2d = pl.pallas_call(
        _make_final_kernel(M, eps),
        out_shape=jax.ShapeDtypeStruct((M, C), jnp.float32),
        grid=(nt,),
        in_specs=[pl.BlockSpec((tm, oc), lambda i: (i, 0)),
                  pl.BlockSpec((N, 8, oc), lambda i: (0, 0, 0)),
                  pl.BlockSpec((1, oc), lambda i: (0, 0)),
                  pl.BlockSpec((1, oc), lambda i: (0, 0)),
                  pl.BlockSpec((oc, C), lambda i: (0, 0)),
                  pl.BlockSpec((nt, 8, oc), lambda i: (0, 0, 0)),
                  pl.BlockSpec((nt, oc, oc), lambda i: (0, 0, 0)),
                  pl.BlockSpec((1, C), lambda i: (0, 0)),
                  pl.BlockSpec((1, C), lambda i: (0, 0)),
                  pl.BlockSpec((tm, C), lambda i: (i, 0))],
        out_specs=pl.BlockSpec((tm, C), lambda i: (i, 0)),
        compiler_params=cp,
    )(y2f, st2, g2r, b2r, w3t, st3, gram, g3r, b3r, x2d)

    out = out2d.reshape(N, H, W, C)
    return jnp.transpose(out, (0, 3, 1, 2))
